# Initial kernel scaffold; baseline (speedup 1.0000x reference)
import jax
import jax.numpy as jnp
from jax.experimental import pallas as pl


def kernel(obj_reg_cls_heads, batch_anchors):
    raise NotImplementedError("write your pallas kernel here")


# speedup vs baseline: 1.4588x; 1.4588x over previous
import jax
import jax.numpy as jnp
from jax.experimental import pallas as pl

TOPN = 1000
MIN_SCORE = 0.05
NMS_THR = 0.5
MAX_OBJ = 100

L = 3
B = 4
N = 17328


def _decode_kernel(h_ref, obj_ref, r0_ref, r1_ref, r2_ref, r3_ref,
                   a0_ref, a1_ref, a2_ref, a3_ref, a4_ref,
                   s_ref, c_ref, x1_ref, y1_ref, x2_ref, y2_ref):
    h = h_ref[0]
    cls = h[:, 5:85]
    m = jnp.max(cls, axis=1)
    iota = jax.lax.broadcasted_iota(jnp.int32, cls.shape, 1)
    c = jnp.min(jnp.where(cls == m[:, None], iota, 10_000), axis=1)
    obj = obj_ref[0, 0, 0]
    s = m * obj
    a0 = a0_ref[0, 0, 0]; a1 = a1_ref[0, 0, 0]; a2 = a2_ref[0, 0, 0]
    a3 = a3_ref[0, 0, 0]; a4 = a4_ref[0, 0, 0]
    cx = (r0_ref[0, 0, 0] + a0) * a4
    cy = (r1_ref[0, 0, 0] + a1) * a4
    w = r2_ref[0, 0, 0] * a2 * a4
    hh = r3_ref[0, 0, 0] * a3 * a4
    s_ref[0, 0, 0] = s
    c_ref[0, 0, 0] = c.astype(jnp.float32)
    x1_ref[0, 0, 0] = (cx - w * 0.5).astype(jnp.int32).astype(jnp.float32)
    y1_ref[0, 0, 0] = (cy - hh * 0.5).astype(jnp.int32).astype(jnp.float32)
    x2_ref[0, 0, 0] = (cx + w * 0.5).astype(jnp.int32).astype(jnp.float32)
    y2_ref[0, 0, 0] = (cy + hh * 0.5).astype(jnp.int32).astype(jnp.float32)


def _decode(obj_reg_cls_heads, batch_anchors):
    NCH = 6
    BS = N // NCH
    h = obj_reg_cls_heads.reshape(L * B, N, 85)
    a = batch_anchors.reshape(L * B, N, 5)
    obj = h[:, :, 0].reshape(L * B, NCH, 1, BS)
    regs = [h[:, :, 1 + i].reshape(L * B, NCH, 1, BS) for i in range(4)]
    ancs = [a[:, :, i].reshape(L * B, NCH, 1, BS) for i in range(5)]

    row_spec = pl.BlockSpec((1, 1, 1, BS), lambda i, j: (i, j, 0, 0))
    outs = pl.pallas_call(
        _decode_kernel,
        grid=(L * B, NCH),
        in_specs=[pl.BlockSpec((1, BS, 85), lambda i, j: (i, j, 0))]
        + [row_spec] * 10,
        out_specs=[row_spec] * 6,
        out_shape=[jax.ShapeDtypeStruct((L * B, NCH, 1, BS), jnp.float32)] * 6,
    )(h, obj, *regs, *ancs)
    def to_img(t):
        return t.reshape(L, B, N).transpose(1, 0, 2).reshape(B, L * N)
    return tuple(to_img(t) for t in outs)


NPAD = 1024


def _nms_kernel(sc_ref, cl_ref, x1_ref, y1_ref, x2_ref, y2_ref,
                os_ref, oc_ref, ob_ref, s_mat):
    sc = sc_ref[0, 0]
    x1 = x1_ref[0, 0]
    y1 = y1_ref[0, 0]
    x2 = x2_ref[0, 0]
    y2 = y2_ref[0, 0]
    areas = jnp.clip((x2 - x1) * (y2 - y1), 0.0001, None)

    colx1 = x1.reshape(1, 8, 128)
    coly1 = y1.reshape(1, 8, 128)
    colx2 = x2.reshape(1, 8, 128)
    coly2 = y2.reshape(1, 8, 128)
    colar = areas.reshape(1, 8, 128)
    colidx = jax.lax.broadcasted_iota(jnp.int32, (1, 8, 128), 1) * 128 + \
        jax.lax.broadcasted_iota(jnp.int32, (1, 8, 128), 2)
    for r in range(NPAD // 128):
        sl = slice(r * 128, (r + 1) * 128)
        rx1 = x1[sl].reshape(128, 1, 1)
        ry1 = y1[sl].reshape(128, 1, 1)
        rx2 = x2[sl].reshape(128, 1, 1)
        ry2 = y2[sl].reshape(128, 1, 1)
        rar = areas[sl].reshape(128, 1, 1)
        ridx = jax.lax.broadcasted_iota(jnp.int32, (128, 1, 1), 0) + r * 128
        sx = jnp.clip(jnp.minimum(rx2, colx2) - jnp.maximum(rx1, colx1), 0.0, None)
        sy = jnp.clip(jnp.minimum(ry2, coly2) - jnp.maximum(ry1, coly1), 0.0, None)
        overlap = sx * sy
        union = jnp.clip(rar + colar - overlap, 0.0001, None)
        iou = overlap / union
        sup = (iou >= NMS_THR) & (colidx > ridx)
        s_mat[sl] = sup.astype(jnp.float32)

    alive0 = (sc > MIN_SCORE).astype(jnp.float32).reshape(8, 128)
    flat = jax.lax.broadcasted_iota(jnp.int32, (8, 128), 0) * 128 + \
        jax.lax.broadcasted_iota(jnp.int32, (8, 128), 1)

    def body(i, al):
        row = s_mat[i]
        ai = jnp.max(jnp.where(flat == i, al, 0.0))
        return al * (1.0 - ai * row)

    kept2 = jax.lax.fori_loop(0, TOPN, body, alive0)

    x = kept2
    for sh in (1, 2, 4, 8, 16, 32, 64):
        x = x + jnp.pad(x, ((0, 0), (sh, 0)))[:, :128]
    row_tot = x[:, 127:128]
    y = row_tot
    for sh in (1, 2, 4):
        y = y + jnp.pad(y, ((sh, 0), (0, 0)))[:8, :]
    rank2 = x + (y - row_tot) - kept2

    kept = kept2.reshape(NPAD)
    rank = rank2.reshape(NPAD)
    pos = jnp.where((kept > 0.0) & (rank < MAX_OBJ), rank, jnp.float32(MAX_OBJ))
    posi = pos.astype(jnp.int32)
    slot = jax.lax.broadcasted_iota(jnp.int32, (MAX_OBJ, NPAD), 0)
    m = (posi.reshape(1, NPAD) == slot).astype(jnp.float32)
    hit = jnp.sum(m, axis=1)
    out_s = jnp.sum(m * sc.reshape(1, NPAD), axis=1) - (1.0 - hit)
    out_c = jnp.sum(m * cl_ref[0, 0].reshape(1, NPAD), axis=1) - (1.0 - hit)
    ox1 = jnp.sum(m * x1.reshape(1, NPAD), axis=1)
    oy1 = jnp.sum(m * y1.reshape(1, NPAD), axis=1)
    ox2 = jnp.sum(m * x2.reshape(1, NPAD), axis=1)
    oy2 = jnp.sum(m * y2.reshape(1, NPAD), axis=1)
    os_ref[0, 0] = out_s
    oc_ref[0, 0] = out_c
    ob_ref[0] = jnp.stack([ox1, oy1, ox2, oy2], axis=1)


def _nms(sc, cl, x1, y1, x2, y2):
    from jax.experimental.pallas import tpu as pltpu
    B_ = sc.shape[0]
    vec = pl.BlockSpec((1, 1, NPAD), lambda i: (i, 0, 0))
    ovec = pl.BlockSpec((1, 1, MAX_OBJ), lambda i: (i, 0, 0))
    obox = pl.BlockSpec((1, MAX_OBJ, 4), lambda i: (i, 0, 0))
    r3 = lambda t: t.reshape(B_, 1, NPAD)
    return pl.pallas_call(
        _nms_kernel,
        grid=(B_,),
        in_specs=[vec] * 6,
        out_specs=[ovec, ovec, obox],
        out_shape=[
            jax.ShapeDtypeStruct((B_, 1, MAX_OBJ), jnp.float32),
            jax.ShapeDtypeStruct((B_, 1, MAX_OBJ), jnp.float32),
            jax.ShapeDtypeStruct((B_, MAX_OBJ, 4), jnp.float32),
        ],
        scratch_shapes=[pltpu.VMEM((NPAD, 8, 128), jnp.float32)],
    )(r3(sc), r3(cl), r3(x1), r3(y1), r3(x2), r3(y2))


def kernel(obj_reg_cls_heads, batch_anchors):
    s, c, x1, y1, x2, y2 = _decode(obj_reg_cls_heads, batch_anchors)
    masked = jnp.where(s > MIN_SCORE, s, jnp.float32(-1.0))
    topv, topi = jax.lax.top_k(masked, TOPN)

    def pad(t):
        return jnp.pad(t, ((0, 0), (0, NPAD - TOPN)))

    sc = jnp.pad(topv, ((0, 0), (0, NPAD - TOPN)), constant_values=-1.0)
    cl = pad(jnp.take_along_axis(c, topi, axis=1))
    gx1 = pad(jnp.take_along_axis(x1, topi, axis=1))
    gy1 = pad(jnp.take_along_axis(y1, topi, axis=1))
    gx2 = pad(jnp.take_along_axis(x2, topi, axis=1))
    gy2 = pad(jnp.take_along_axis(y2, topi, axis=1))

    out_s, out_c, out_b = _nms(sc, cl, gx1, gy1, gx2, gy2)
    return out_s[:, 0, :], out_c[:, 0, :], out_b
